# baseline (speedup 1.0000x reference)
import functools

import jax
import jax.numpy as jnp
from jax import lax
from jax.experimental import pallas as pl
from jax.experimental.pallas import tpu as pltpu
from jax.experimental.pallas import tpu_sc as plsc

_D = 64
_NC = 2
_NS = 16
_NW = _NC * _NS
_NBUF = 4
_LOOK = _NBUF - 1


def _gather_kernel(n_text, seq):
    rows_per_w = n_text // _NW
    mesh = plsc.VectorSubcoreMesh(core_axis_name="c", subcore_axis_name="s")

    @functools.partial(
        pl.kernel,
        out_type=jax.ShapeDtypeStruct((n_text * seq, 2 * _D), jnp.float32),
        mesh=mesh,
        scratch_types=[
            pltpu.VMEM((rows_per_w, seq), jnp.int32),
            pltpu.VMEM((_NBUF, seq, _D), jnp.float32),
        ]
        + [pltpu.SemaphoreType.DMA] * (2 * _NBUF),
        compiler_params=pltpu.CompilerParams(use_tc_tiling_on_sc=False),
    )
    def k(idx_hbm, table_hbm, out_hbm, idx_v, rows, *sems):
        sem_g = sems[:_NBUF]
        sem_s = sems[_NBUF:]
        wid = lax.axis_index("s") * _NC + lax.axis_index("c")
        base = wid * rows_per_w

        pltpu.sync_copy(idx_hbm.at[pl.ds(base, rows_per_w)], idx_v)

        def gather(i, b):
            return pltpu.make_async_copy(
                table_hbm.at[idx_v.at[i]], rows.at[b], sem_g[b])

        def store(i, b):
            return pltpu.make_async_copy(
                rows.at[b],
                out_hbm.at[pl.ds((base + i) * seq, seq), pl.ds(0, _D)],
                sem_s[b])

        def step(i, b, drain, prefetch):
            gather(i, b).wait()
            store(i, b).start()
            bp = (b + _LOOK) % _NBUF
            if drain:
                store(i - 1, bp).wait()
            if prefetch:
                gather(i + _LOOK, bp).start()

        for b in range(_LOOK):
            gather(b, b).start()
        for b in range(_NBUF):
            step(b, b, drain=(b >= 1), prefetch=True)

        n_groups = rows_per_w // _NBUF

        def group(g):
            for b in range(_NBUF):
                step(g * _NBUF + b, b, drain=True, prefetch=True)

        pl.loop(1, n_groups - 1)(group)

        i0 = (n_groups - 1) * _NBUF
        for b in range(_NBUF):
            i = i0 + b
            ok = i + _LOOK < rows_per_w
            step(i, b, drain=ok, prefetch=ok)

        for b in range(_NBUF):
            store(i0 + b, b).wait()

    return k


_TC_CHUNK = 512


def _transpose_kernel(vocab):
    grid = (vocab + _TC_CHUNK - 1) // _TC_CHUNK

    def body(wt_ref, out_ref):
        out_ref[:, :_D] = wt_ref[...].T

    return pl.pallas_call(
        body,
        grid=(grid,),
        in_specs=[pl.BlockSpec((_D, _TC_CHUNK), lambda c: (0, c))],
        out_specs=pl.BlockSpec((_TC_CHUNK, 2 * _D), lambda c: (c, 0)),
        out_shape=jax.ShapeDtypeStruct((vocab, 2 * _D), jnp.float32),
    )


def kernel(text, embedding_weight):
    n_text, seq = text.shape
    vocab = embedding_weight.shape[0]
    w128 = _transpose_kernel(vocab)(embedding_weight.T)
    w2 = w128.reshape(2 * vocab, _D)
    out = _gather_kernel(n_text, seq)(text * 2, w2)
    return out.reshape(n_text, seq, 2 * _D)[:, :, :_D]


# speedup vs baseline: 2.2639x; 2.2639x over previous
import functools

import jax
import jax.numpy as jnp
from jax import lax
from jax.experimental import pallas as pl
from jax.experimental.pallas import tpu as pltpu
from jax.experimental.pallas import tpu_sc as plsc

_D = 64
_NC = 2
_NS = 16
_NW = _NC * _NS
_NBUF = 4
_LOOK = _NBUF - 1


def _gather_kernel(n_text, seq):
    rows_per_w = n_text // _NW
    mesh = plsc.VectorSubcoreMesh(core_axis_name="c", subcore_axis_name="s")

    @functools.partial(
        pl.kernel,
        out_type=jax.ShapeDtypeStruct((n_text * seq, 2 * _D), jnp.float32),
        mesh=mesh,
        scratch_types=[
            pltpu.VMEM((rows_per_w, seq), jnp.int32),
            pltpu.VMEM((_NBUF, seq, _D), jnp.float32),
        ]
        + [pltpu.SemaphoreType.DMA] * (2 * _NBUF),
        compiler_params=pltpu.CompilerParams(use_tc_tiling_on_sc=False),
    )
    def k(idx_hbm, table_hbm, out_hbm, idx_v, rows, *sems):
        sem_g = sems[:_NBUF]
        sem_s = sems[_NBUF:]
        wid = lax.axis_index("s") * _NC + lax.axis_index("c")
        base = wid * rows_per_w

        pltpu.sync_copy(idx_hbm.at[pl.ds(base, rows_per_w)], idx_v)

        def gather(i, b):
            return pltpu.make_async_copy(
                table_hbm.at[idx_v.at[i]], rows.at[b], sem_g[b])

        def store(i, b):
            return pltpu.make_async_copy(
                rows.at[b],
                out_hbm.at[pl.ds((base + i) * seq, seq), pl.ds(0, _D)],
                sem_s[b])

        def step(i, b, drain, prefetch):
            gather(i, b).wait()
            store(i, b).start()
            bp = (b + _LOOK) % _NBUF
            if drain:
                store(i - 1, bp).wait()
            if prefetch:
                gather(i + _LOOK, bp).start()

        for b in range(_LOOK):
            gather(b, b).start()
        for b in range(_NBUF):
            step(b, b, drain=(b >= 1), prefetch=True)

        n_groups = rows_per_w // _NBUF

        def group(g):
            for b in range(_NBUF):
                step(g * _NBUF + b, b, drain=True, prefetch=True)

        pl.loop(1, n_groups - 1)(group)

        i0 = (n_groups - 1) * _NBUF
        for b in range(_NBUF):
            i = i0 + b
            ok = i + _LOOK < rows_per_w
            step(i, b, drain=ok, prefetch=ok)

        for b in range(_NBUF):
            store(i0 + b, b).wait()

    return k


_TC_CHUNK = 4096


def _transpose_kernel(vocab):
    grid = (vocab + _TC_CHUNK - 1) // _TC_CHUNK

    def body(wt_ref, out_ref):
        out_ref[:, :_D] = wt_ref[...].T

    return pl.pallas_call(
        body,
        grid=(grid,),
        in_specs=[pl.BlockSpec((_D, _TC_CHUNK), lambda c: (0, c))],
        out_specs=pl.BlockSpec((_TC_CHUNK, 2 * _D), lambda c: (c, 0)),
        out_shape=jax.ShapeDtypeStruct((vocab, 2 * _D), jnp.float32),
    )


def kernel(text, embedding_weight):
    n_text, seq = text.shape
    vocab = embedding_weight.shape[0]
    w128 = _transpose_kernel(vocab)(embedding_weight.T)
    w2 = w128.reshape(2 * vocab, _D)
    out = _gather_kernel(n_text, seq)(text * 2, w2)
    return out.reshape(n_text, seq, 2 * _D)[:, :, :_D]


# speedup vs baseline: 2.2824x; 1.0082x over previous
import functools

import jax
import jax.numpy as jnp
from jax import lax
from jax.experimental import pallas as pl
from jax.experimental.pallas import tpu as pltpu
from jax.experimental.pallas import tpu_sc as plsc

_D = 64
_NC = 2
_NS = 16
_NW = _NC * _NS
_NBUF = 4
_LOOK = _NBUF - 1


def _gather_kernel(n_text, seq):
    rows_per_w = n_text // _NW
    mesh = plsc.VectorSubcoreMesh(core_axis_name="c", subcore_axis_name="s")

    @functools.partial(
        pl.kernel,
        out_type=jax.ShapeDtypeStruct((n_text * seq, 2 * _D), jnp.float32),
        mesh=mesh,
        scratch_types=[
            pltpu.VMEM((rows_per_w, seq), jnp.int32),
            pltpu.VMEM((_NBUF, seq, _D), jnp.float32),
        ]
        + [pltpu.SemaphoreType.DMA] * (2 * _NBUF),
        compiler_params=pltpu.CompilerParams(use_tc_tiling_on_sc=False),
    )
    def k(idx_hbm, table_hbm, out_hbm, idx_v, rows, *sems):
        sem_g = sems[:_NBUF]
        sem_s = sems[_NBUF:]
        wid = lax.axis_index("s") * _NC + lax.axis_index("c")
        base = wid * rows_per_w

        pltpu.sync_copy(idx_hbm.at[pl.ds(base, rows_per_w)], idx_v)

        def gather(i, b):
            return pltpu.make_async_copy(
                table_hbm.at[idx_v.at[i]], rows.at[b], sem_g[b])

        def store(i, b):
            return pltpu.make_async_copy(
                rows.at[b],
                out_hbm.at[pl.ds((base + i) * seq, seq), pl.ds(0, _D)],
                sem_s[b])

        def step(i, b, drain, prefetch):
            gather(i, b).wait()
            store(i, b).start()
            bp = (b + _LOOK) % _NBUF
            if drain:
                store(i - 1, bp).wait()
            if prefetch:
                gather(i + _LOOK, bp).start()

        for b in range(_LOOK):
            gather(b, b).start()
        for b in range(_NBUF):
            step(b, b, drain=(b >= 1), prefetch=True)

        n_groups = rows_per_w // _NBUF

        def group(g):
            for b in range(_NBUF):
                step(g * _NBUF + b, b, drain=True, prefetch=True)

        pl.loop(1, n_groups - 1)(group)

        i0 = (n_groups - 1) * _NBUF
        for b in range(_NBUF):
            i = i0 + b
            ok = i + _LOOK < rows_per_w
            step(i, b, drain=ok, prefetch=ok)

        for b in range(_NBUF):
            store(i0 + b, b).wait()

    return k


_TC_CHUNK = 4096


def _transpose_kernel(vocab):
    grid = (vocab + _TC_CHUNK - 1) // _TC_CHUNK

    half = _TC_CHUNK // 2

    def body(wt_ref, out_ref):
        x = wt_ref[...]
        out_ref[:, :_D] = x[:, :half].T
        out_ref[:, _D:] = x[:, half:].T

    return pl.pallas_call(
        body,
        grid=(grid,),
        in_specs=[pl.BlockSpec((_D, _TC_CHUNK), lambda c: (0, c))],
        out_specs=pl.BlockSpec((half, 2 * _D), lambda c: (c, 0)),
        out_shape=jax.ShapeDtypeStruct((grid * half, 2 * _D), jnp.float32),
    )


def kernel(text, embedding_weight):
    n_text, seq = text.shape
    vocab = embedding_weight.shape[0]
    grid = (vocab + _TC_CHUNK - 1) // _TC_CHUNK
    w128 = _transpose_kernel(vocab)(embedding_weight.T)
    w2 = w128.reshape(grid * _TC_CHUNK, _D)
    half = _TC_CHUNK // 2
    r = text % _TC_CHUNK
    idxp = (text - r) + 2 * (r % half) + (r // half)
    out = _gather_kernel(n_text, seq)(idxp, w2)
    return out.reshape(n_text, seq, 2 * _D)[:, :, :_D]


# speedup vs baseline: 2.5387x; 1.1123x over previous
import functools

import jax
import jax.numpy as jnp
from jax import lax
from jax.experimental import pallas as pl
from jax.experimental.pallas import tpu as pltpu
from jax.experimental.pallas import tpu_sc as plsc

_D = 64
_NC = 2
_NS = 16
_NW = _NC * _NS
_NBUF = 4
_LOOK = _NBUF - 1


def _gather_kernel(n_text, seq):
    rows_per_w = n_text // _NW
    mesh = plsc.VectorSubcoreMesh(core_axis_name="c", subcore_axis_name="s")

    @functools.partial(
        pl.kernel,
        out_type=jax.ShapeDtypeStruct((n_text * seq, 2 * _D), jnp.float32),
        mesh=mesh,
        scratch_types=[
            pltpu.VMEM((rows_per_w, seq), jnp.int32),
            pltpu.VMEM((_NBUF, seq, _D), jnp.float32),
        ]
        + [pltpu.SemaphoreType.DMA] * (2 * _NBUF),
        compiler_params=pltpu.CompilerParams(use_tc_tiling_on_sc=False),
    )
    def k(idx_hbm, table_hbm, out_hbm, idx_v, rows, *sems):
        sem_g = sems[:_NBUF]
        sem_s = sems[_NBUF:]
        wid = lax.axis_index("s") * _NC + lax.axis_index("c")
        base = wid * rows_per_w

        pltpu.sync_copy(idx_hbm.at[pl.ds(base, rows_per_w)], idx_v)

        def gather(i, b):
            return pltpu.make_async_copy(
                table_hbm.at[idx_v.at[i]], rows.at[b], sem_g[b])

        def store(i, b):
            return pltpu.make_async_copy(
                rows.at[b],
                out_hbm.at[pl.ds((base + i) * seq, seq), pl.ds(0, _D)],
                sem_s[b])

        def step(i, b, drain, prefetch):
            gather(i, b).wait()
            store(i, b).start()
            bp = (b + _LOOK) % _NBUF
            if drain:
                store(i - 1, bp).wait()
            if prefetch:
                gather(i + _LOOK, bp).start()

        for b in range(_LOOK):
            gather(b, b).start()
        for b in range(_NBUF):
            step(b, b, drain=(b >= 1), prefetch=True)

        n_groups = rows_per_w // _NBUF

        def group(g):
            for b in range(_NBUF):
                step(g * _NBUF + b, b, drain=True, prefetch=True)

        pl.loop(1, n_groups - 1)(group)

        i0 = (n_groups - 1) * _NBUF
        for b in range(_NBUF):
            i = i0 + b
            ok = i + _LOOK < rows_per_w
            step(i, b, drain=ok, prefetch=ok)

        for b in range(_NBUF):
            store(i0 + b, b).wait()

    return k


_TC_CHUNK = 8192


def _transpose_kernel(vocab):
    grid = (vocab + _TC_CHUNK - 1) // _TC_CHUNK

    half = _TC_CHUNK // 2

    def body(wt_ref, out_ref):
        x = wt_ref[...]
        out_ref[:, :_D] = x[:, :half].T
        out_ref[:, _D:] = x[:, half:].T

    return pl.pallas_call(
        body,
        grid=(grid,),
        in_specs=[pl.BlockSpec((_D, _TC_CHUNK), lambda c: (0, c))],
        out_specs=pl.BlockSpec((half, 2 * _D), lambda c: (c, 0)),
        out_shape=jax.ShapeDtypeStruct((grid * half, 2 * _D), jnp.float32),
    )


def kernel(text, embedding_weight):
    n_text, seq = text.shape
    vocab = embedding_weight.shape[0]
    grid = (vocab + _TC_CHUNK - 1) // _TC_CHUNK
    w128 = _transpose_kernel(vocab)(embedding_weight.T)
    w2 = w128.reshape(grid * _TC_CHUNK, _D)
    half = _TC_CHUNK // 2
    r = text % _TC_CHUNK
    idxp = (text - r) + 2 * (r % half) + (r // half)
    out = _gather_kernel(n_text, seq)(idxp, w2)
    return out.reshape(n_text, seq, 2 * _D)[:, :, :_D]


# speedup vs baseline: 2.6898x; 1.0595x over previous
import functools

import jax
import jax.numpy as jnp
from jax import lax
from jax.experimental import pallas as pl
from jax.experimental.pallas import tpu as pltpu
from jax.experimental.pallas import tpu_sc as plsc

_D = 64
_NC = 2
_NS = 16
_NW = _NC * _NS
_NBUF = 4
_LOOK = _NBUF - 1


def _gather_kernel(n_text, seq):
    rows_per_w = n_text // _NW
    mesh = plsc.VectorSubcoreMesh(core_axis_name="c", subcore_axis_name="s")

    @functools.partial(
        pl.kernel,
        out_type=jax.ShapeDtypeStruct((n_text * seq, 2 * _D), jnp.float32),
        mesh=mesh,
        scratch_types=[
            pltpu.VMEM((rows_per_w, seq), jnp.int32),
            pltpu.VMEM((_NBUF, seq, _D), jnp.float32),
        ]
        + [pltpu.SemaphoreType.DMA] * (2 * _NBUF),
        compiler_params=pltpu.CompilerParams(use_tc_tiling_on_sc=False),
    )
    def k(idx_hbm, table_hbm, out_hbm, idx_v, rows, *sems):
        sem_g = sems[:_NBUF]
        sem_s = sems[_NBUF:]
        wid = lax.axis_index("s") * _NC + lax.axis_index("c")
        base = wid * rows_per_w

        pltpu.sync_copy(idx_hbm.at[pl.ds(base, rows_per_w)], idx_v)

        def gather(i, b):
            return pltpu.make_async_copy(
                table_hbm.at[idx_v.at[i]], rows.at[b], sem_g[b])

        def store(i, b):
            return pltpu.make_async_copy(
                rows.at[b],
                out_hbm.at[pl.ds((base + i) * seq, seq), pl.ds(0, _D)],
                sem_s[b])

        def step(i, b, drain, prefetch):
            gather(i, b).wait()
            store(i, b).start()
            bp = (b + _LOOK) % _NBUF
            if drain:
                store(i - 1, bp).wait()
            if prefetch:
                gather(i + _LOOK, bp).start()

        for b in range(_LOOK):
            gather(b, b).start()
        for b in range(_NBUF):
            step(b, b, drain=(b >= 1), prefetch=True)

        n_groups = rows_per_w // _NBUF

        def group(g):
            for b in range(_NBUF):
                step(g * _NBUF + b, b, drain=True, prefetch=True)

        pl.loop(1, n_groups - 1)(group)

        i0 = (n_groups - 1) * _NBUF
        for b in range(_NBUF):
            i = i0 + b
            ok = i + _LOOK < rows_per_w
            step(i, b, drain=ok, prefetch=ok)

        for b in range(_NBUF):
            store(i0 + b, b).wait()

    return k


_TC_CHUNK = 16384


def _transpose_kernel(vocab):
    grid = (vocab + _TC_CHUNK - 1) // _TC_CHUNK

    half = _TC_CHUNK // 2

    def body(wt_ref, out_ref):
        x = wt_ref[...]
        out_ref[:, :_D] = x[:, :half].T
        out_ref[:, _D:] = x[:, half:].T

    return pl.pallas_call(
        body,
        grid=(grid,),
        in_specs=[pl.BlockSpec((_D, _TC_CHUNK), lambda c: (0, c))],
        out_specs=pl.BlockSpec((half, 2 * _D), lambda c: (c, 0)),
        out_shape=jax.ShapeDtypeStruct((grid * half, 2 * _D), jnp.float32),
    )


def kernel(text, embedding_weight):
    n_text, seq = text.shape
    vocab = embedding_weight.shape[0]
    grid = (vocab + _TC_CHUNK - 1) // _TC_CHUNK
    w128 = _transpose_kernel(vocab)(embedding_weight.T)
    w2 = w128.reshape(grid * _TC_CHUNK, _D)
    half = _TC_CHUNK // 2
    r = text % _TC_CHUNK
    idxp = (text - r) + 2 * (r % half) + (r // half)
    out = _gather_kernel(n_text, seq)(idxp, w2)
    return out.reshape(n_text, seq, 2 * _D)[:, :, :_D]


# speedup vs baseline: 2.7566x; 1.0249x over previous
import functools

import jax
import jax.numpy as jnp
from jax import lax
from jax.experimental import pallas as pl
from jax.experimental.pallas import tpu as pltpu
from jax.experimental.pallas import tpu_sc as plsc

_D = 64
_NC = 2
_NS = 16
_NW = _NC * _NS
_NBUF = 4
_LOOK = _NBUF - 1


def _gather_kernel(n_text, seq):
    rows_per_w = n_text // _NW
    mesh = plsc.VectorSubcoreMesh(core_axis_name="c", subcore_axis_name="s")

    @functools.partial(
        pl.kernel,
        out_type=jax.ShapeDtypeStruct((n_text * seq, 2 * _D), jnp.float32),
        mesh=mesh,
        scratch_types=[
            pltpu.VMEM((rows_per_w, seq), jnp.int32),
            pltpu.VMEM((_NBUF, seq, _D), jnp.float32),
        ]
        + [pltpu.SemaphoreType.DMA] * (2 * _NBUF),
        compiler_params=pltpu.CompilerParams(use_tc_tiling_on_sc=False),
    )
    def k(idx_hbm, table_hbm, out_hbm, idx_v, rows, *sems):
        sem_g = sems[:_NBUF]
        sem_s = sems[_NBUF:]
        wid = lax.axis_index("s") * _NC + lax.axis_index("c")
        base = wid * rows_per_w

        pltpu.sync_copy(idx_hbm.at[pl.ds(base, rows_per_w)], idx_v)

        def gather(i, b):
            return pltpu.make_async_copy(
                table_hbm.at[idx_v.at[i]], rows.at[b], sem_g[b])

        def store(i, b):
            return pltpu.make_async_copy(
                rows.at[b],
                out_hbm.at[pl.ds((base + i) * seq, seq), pl.ds(0, _D)],
                sem_s[b])

        def step(i, b, drain, prefetch):
            gather(i, b).wait()
            store(i, b).start()
            bp = (b + _LOOK) % _NBUF
            if drain:
                store(i - 1, bp).wait()
            if prefetch:
                gather(i + _LOOK, bp).start()

        for b in range(_LOOK):
            gather(b, b).start()
        for b in range(_NBUF):
            step(b, b, drain=(b >= 1), prefetch=True)

        n_groups = rows_per_w // _NBUF

        def group(g):
            for b in range(_NBUF):
                step(g * _NBUF + b, b, drain=True, prefetch=True)

        pl.loop(1, n_groups - 1)(group)

        i0 = (n_groups - 1) * _NBUF
        for b in range(_NBUF):
            i = i0 + b
            ok = i + _LOOK < rows_per_w
            step(i, b, drain=ok, prefetch=ok)

        for b in range(_NBUF):
            store(i0 + b, b).wait()

    return k


_TC_CHUNK = 32768


def _transpose_kernel(vocab):
    grid = (vocab + _TC_CHUNK - 1) // _TC_CHUNK

    half = _TC_CHUNK // 2

    def body(wt_ref, out_ref):
        x = wt_ref[...]
        out_ref[:, :_D] = x[:, :half].T
        out_ref[:, _D:] = x[:, half:].T

    return pl.pallas_call(
        body,
        grid=(grid,),
        in_specs=[pl.BlockSpec((_D, _TC_CHUNK), lambda c: (0, c))],
        out_specs=pl.BlockSpec((half, 2 * _D), lambda c: (c, 0)),
        out_shape=jax.ShapeDtypeStruct((grid * half, 2 * _D), jnp.float32),
    )


def kernel(text, embedding_weight):
    n_text, seq = text.shape
    vocab = embedding_weight.shape[0]
    grid = (vocab + _TC_CHUNK - 1) // _TC_CHUNK
    w128 = _transpose_kernel(vocab)(embedding_weight.T)
    w2 = w128.reshape(grid * _TC_CHUNK, _D)
    half = _TC_CHUNK // 2
    r = text % _TC_CHUNK
    idxp = (text - r) + 2 * (r % half) + (r // half)
    out = _gather_kernel(n_text, seq)(idxp, w2)
    return out.reshape(n_text, seq, 2 * _D)[:, :, :_D]
